# baseline (speedup 1.0000x reference)
import jax
import jax.numpy as jnp
from jax.experimental import pallas as pl
from jax.experimental.pallas import tpu as pltpu

_PRE_NMS_THRESH = 0.05
_PRE_NMS_TOP_N = 1000
_NMS_THRESH = 0.5
_POST_TOP_N = 100
_K = 1024


def _score_kernel(cls_ref, conf_ref, out_ref):
    s = jax.nn.sigmoid(cls_ref[0])
    cf = jax.nn.sigmoid(conf_ref[0])
    out_ref[0] = jnp.where(s > _PRE_NMS_THRESH, s * cf, 0.0)


def _decode(r0, r1, r2, r3, c0, c1, l0, l1):
    pw = r0 + r1
    ph = r2 + r3
    cx = l0 + c0
    cy = l1 + c1
    x1 = cx - pw / 2.0
    y1 = cy - ph / 2.0
    x2 = cx + pw / 2.0
    y2 = cy + ph / 2.0
    p1x = x1 + r0
    p1y = y1
    p2x = x2
    p2y = y1 + r2
    p3x = x2 - r0
    p3y = y2
    p4x = x1
    p4y = y2 - r2
    cxm = ((p1x + p2x) + (p3x + p4x)) * 0.25
    cym = ((p1y + p2y) + (p3y + p4y)) * 0.25
    ex = p2x - p1x
    ey = p2y - p1y
    rr = jnp.sqrt(ex * ex + ey * ey)
    pos = rr > 0.0
    rs = jnp.where(pos, rr, 1.0)
    ca = jnp.where(pos, ex / rs, 1.0)
    sa = jnp.where(pos, -ey / rs, 0.0)
    d1x = p1x - cxm
    d1y = p1y - cym
    d2x = p2x - cxm
    d2y = p2y - cym
    d3x = p3x - cxm
    d3y = p3y - cym
    d4x = p4x - cxm
    d4y = p4y - cym
    rx1 = ca * d1x - sa * d1y
    rx2 = ca * d2x - sa * d2y
    rx3 = ca * d3x - sa * d3y
    rx4 = ca * d4x - sa * d4y
    ry1 = sa * d1x + ca * d1y
    ry2 = sa * d2x + ca * d2y
    ry3 = sa * d3x + ca * d3y
    ry4 = sa * d4x + ca * d4y
    wr = jnp.maximum(jnp.maximum(rx1, rx2), jnp.maximum(rx3, rx4)) - \
        jnp.minimum(jnp.minimum(rx1, rx2), jnp.minimum(rx3, rx4))
    hr = jnp.maximum(jnp.maximum(ry1, ry2), jnp.maximum(ry3, ry4)) - \
        jnp.minimum(jnp.minimum(ry1, ry2), jnp.minimum(ry3, ry4))
    caa = jnp.abs(ca)
    saa = jnp.abs(sa)
    exh = (wr * caa + hr * saa) / 2.0
    eyh = (wr * saa + hr * caa) / 2.0
    bx1 = cxm - exh
    by1 = cym - eyh
    bx2 = cxm + exh
    by2 = cym + eyh
    area = jnp.maximum(bx2 - bx1, 0.0) * jnp.maximum(by2 - by1, 0.0)
    return cxm, cym, wr, hr, ex, ey, bx1, by1, bx2, by2, area


def _nms_kernel(data_ref, data_t_ref, vals_ref, vals_t_ref,
                cls_ref, cls_t_ref, out_ref, sup_ref):
    d = data_ref[0]
    dt = data_t_ref[0]
    row = _decode(d[0:1], d[1:2], d[2:3], d[3:4],
                  d[4:5], d[5:6], d[6:7], d[7:8])
    col = _decode(dt[:, 0:1], dt[:, 1:2], dt[:, 2:3], dt[:, 3:4],
                  dt[:, 4:5], dt[:, 5:6], dt[:, 6:7], dt[:, 7:8])
    cxm, cym, wr, hr, ex, ey, bx1r, by1r, bx2r, by2r, arear = row
    _, _, _, _, _, _, bx1c, by1c, bx2c, by2c, areac = col

    vals = vals_ref[0]
    vt = vals_t_ref[0]
    clsr = cls_ref[0]
    clsc = cls_t_ref[0]
    valid_r = vals > 0.0
    valid_c = vt > 0.0

    ix1 = jnp.maximum(bx1c, bx1r)
    iy1 = jnp.maximum(by1c, by1r)
    ix2 = jnp.minimum(bx2c, bx2r)
    iy2 = jnp.minimum(by2c, by2r)
    inter = jnp.maximum(ix2 - ix1, 0.0) * jnp.maximum(iy2 - iy1, 0.0)
    iou = inter / (areac + arear - inter + 1e-9)
    jr = jax.lax.broadcasted_iota(jnp.int32, (1, _K), 1)
    ji = jax.lax.broadcasted_iota(jnp.int32, (_K, 1), 0)
    sup = (iou > _NMS_THRESH) & (clsc == clsr) & (jr > ji) & valid_c
    sup_ref[:, :] = sup.astype(jnp.float32)

    keep0 = valid_r.astype(jnp.float32)

    def body(i, keep):
        srow = sup_ref[pl.ds(i, 1), :]
        live = jnp.max(jnp.where(jr == i, keep, 0.0))
        return keep * (1.0 - srow * live)

    keep = jax.lax.fori_loop(0, _K, body, keep0)

    sc = jnp.sqrt(jnp.maximum(vals, 1e-12)) * valid_r.astype(jnp.float32)
    final = jnp.where(keep > 0.5, sc, -1.0)
    out_ref[0] = jnp.concatenate(
        [cxm, cym, wr, hr, ex, ey, sc, final], axis=0)


def kernel(locations, box_cls, box_regression, center, confs):
    n, c, h, w = box_cls.shape
    hw = h * w
    cls2 = box_cls.reshape(n, c, hw)
    conf2 = confs.reshape(n, 1, hw)
    scores = pl.pallas_call(
        _score_kernel,
        grid=(n,),
        in_specs=[
            pl.BlockSpec((1, c, hw), lambda i: (i, 0, 0)),
            pl.BlockSpec((1, 1, hw), lambda i: (i, 0, 0)),
        ],
        out_specs=pl.BlockSpec((1, c, hw), lambda i: (i, 0, 0)),
        out_shape=jax.ShapeDtypeStruct((n, c, hw), jnp.float32),
    )(cls2, conf2)

    flat = scores.transpose(0, 2, 1).reshape(n, hw * c)
    vals, idx = jax.lax.top_k(flat, _PRE_NMS_TOP_N)
    loc_idx = idx // c
    cls_idx = idx % c

    reg_t = box_regression.reshape(n, 4, hw).transpose(0, 2, 1)
    ctr_t = center.reshape(n, 2, hw).transpose(0, 2, 1)
    g_reg = jnp.take_along_axis(reg_t, loc_idx[:, :, None], axis=1)
    g_ctr = jnp.take_along_axis(ctr_t, loc_idx[:, :, None], axis=1)
    g_loc = locations[loc_idx]
    dat_t = jnp.concatenate([g_reg, g_ctr, g_loc], axis=2)
    pad = _K - _PRE_NMS_TOP_N
    dat_t = jnp.pad(dat_t, ((0, 0), (0, pad), (0, 0)))
    dat = dat_t.transpose(0, 2, 1)
    valsp = jnp.pad(vals, ((0, 0), (0, pad)))
    clsf = jnp.pad(cls_idx.astype(jnp.float32), ((0, 0), (0, pad)))

    out = pl.pallas_call(
        _nms_kernel,
        grid=(n,),
        in_specs=[
            pl.BlockSpec((1, 8, _K), lambda i: (i, 0, 0)),
            pl.BlockSpec((1, _K, 8), lambda i: (i, 0, 0)),
            pl.BlockSpec((1, 1, _K), lambda i: (i, 0, 0)),
            pl.BlockSpec((1, _K, 1), lambda i: (i, 0, 0)),
            pl.BlockSpec((1, 1, _K), lambda i: (i, 0, 0)),
            pl.BlockSpec((1, _K, 1), lambda i: (i, 0, 0)),
        ],
        out_specs=pl.BlockSpec((1, 8, _K), lambda i: (i, 0, 0)),
        out_shape=jax.ShapeDtypeStruct((n, 8, _K), jnp.float32),
        scratch_shapes=[pltpu.VMEM((_K, _K), jnp.float32)],
    )(dat, dat_t, valsp[:, None, :], valsp[:, :, None],
      clsf[:, None, :], clsf[:, :, None])

    cxm = out[:, 0]
    cym = out[:, 1]
    wr = out[:, 2]
    hr = out[:, 3]
    exv = out[:, 4]
    eyv = out[:, 5]
    final = out[:, 7]

    top_sc, top_i = jax.lax.top_k(final, _POST_TOP_N)
    ga = lambda a: jnp.take_along_axis(a, top_i, axis=1)
    angle = jnp.arctan2(ga(eyv), ga(exv))
    boxes = jnp.stack([ga(cxm), ga(cym), ga(wr), ga(hr), angle], axis=2)
    out_scores = jnp.where(top_sc > 0.0, top_sc, 0.0)
    out_arr = jnp.concatenate([boxes, out_scores[:, :, None]], axis=2)
    clsp = jnp.pad(cls_idx, ((0, 0), (0, pad)))
    labels = jnp.where(top_sc > 0.0, ga(clsp), -1)
    return out_arr, labels
